# baseline (speedup 1.0000x reference)
import functools

import jax
import jax.numpy as jnp
from jax import lax
from jax.experimental import pallas as pl
from jax.experimental.pallas import tpu as pltpu
from jax.experimental.pallas import tpu_sc as plsc

NW = 32


def _routing_body(nt, ntok, x_ref, tp_ref, wgi_ref, wgt_ref, bg_ref,
                  laux_ref, w_ref, e_ref, r_ref, cnt16_ref, acc_ref, cnt_ref):
    t = pl.program_id(0)
    logits = (jnp.dot(x_ref[...], wgi_ref[...], preferred_element_type=jnp.float32)
              + jnp.dot(tp_ref[...], wgt_ref[...], preferred_element_type=jnp.float32)
              + bg_ref[...])
    mr, ne = logits.shape
    lane = lax.broadcasted_iota(jnp.int32, logits.shape, 1)
    m1 = jnp.max(logits, axis=1, keepdims=True)
    i1 = jnp.min(jnp.where(logits == m1, lane, ne), axis=1, keepdims=True)
    masked = jnp.where(lane == i1, -jnp.inf, logits)
    m2 = jnp.max(masked, axis=1, keepdims=True)
    i2 = jnp.min(jnp.where(masked == m2, lane, ne), axis=1, keepdims=True)
    s = jnp.exp(m2 - m1)
    w1 = 1.0 / (1.0 + s)
    w2 = s / (1.0 + s)
    sel1 = lane == i1
    sel2 = lane == i2

    @pl.when(t == 0)
    def _():
        acc_ref[...] = jnp.zeros_like(acc_ref)
        cnt_ref[...] = jnp.zeros_like(cnt_ref)

    o0 = sel1.astype(jnp.float32)
    o1 = sel2.astype(jnp.float32)
    row = lax.broadcasted_iota(jnp.int32, (mr, mr), 0)
    col = lax.broadcasted_iota(jnp.int32, (mr, mr), 1)
    tril = (col < row).astype(jnp.float32)
    rk0 = jnp.dot(tril, o0, preferred_element_type=jnp.float32)
    rk1 = jnp.dot(tril, o1, preferred_element_type=jnp.float32)
    base0 = cnt_ref[0:1, :].astype(jnp.float32)
    base1 = cnt_ref[1:2, :].astype(jnp.float32)
    r0 = jnp.sum(jnp.where(sel1, rk0 + base0, 0.0), axis=1, keepdims=True)
    r1 = jnp.sum(jnp.where(sel2, rk1 + base1, 0.0), axis=1, keepdims=True)
    r_ref[...] = jnp.concatenate([r0, r1], axis=1).astype(jnp.int32)
    e_ref[...] = jnp.concatenate([i1, i2], axis=1)
    w_ref[...] = jnp.concatenate([w1, w2], axis=1)
    cnt_ref[0:1, :] += jnp.sum(o0, axis=0, keepdims=True).astype(jnp.int32)
    cnt_ref[1:2, :] += jnp.sum(o1, axis=0, keepdims=True).astype(jnp.int32)

    p = jnp.exp(logits - m1)
    p = p / jnp.sum(p, axis=1, keepdims=True)
    acc_ref[0:1, :] += jnp.sum(p, axis=0, keepdims=True)
    acc_ref[1:2, :] += jnp.sum(o0 + o1, axis=0, keepdims=True)

    @pl.when(t == nt - 1)
    def _():
        laux_ref[...] = (jnp.sum(acc_ref[0:1, :] * acc_ref[1:2, :])
                         / (ntok * ntok)).reshape(1, 1)
        cnt16_ref[...] = jnp.concatenate([cnt_ref[0:1, :], cnt_ref[1:2, :]], axis=1)


def _finalize_body(m, ne, cnt_ref, e_ref, r_ref, pos_ref, meta_ref):
    t = pl.program_id(0)
    e_nk = e_ref[...]
    kk = lax.broadcasted_iota(jnp.int32, e_nk.shape, 1)
    lane64 = lax.broadcasted_iota(jnp.int32, (1, 64), 1)
    add = jnp.zeros(e_nk.shape, jnp.int32)
    te = jnp.zeros((1, 64), jnp.int32)
    po = jnp.int32(0)
    btl = jnp.int32(0)
    for e in range(ne):
        c0 = cnt_ref[e]
        c1 = cnt_ref[ne + e]
        pc = ((c0 + c1 + m - 1) // m) * m
        add = add + jnp.where(e_nk == e, po + jnp.where(kk == 1, c0, 0), 0)
        po = po + pc
        btl = btl + pc // m
        te = te + (lane64 >= btl).astype(jnp.int32)
    pos_ref[...] = r_ref[...] + add
    na = btl

    @pl.when(t == 0)
    def _():
        meta_ref[...] = jnp.where(lane64 == 63, na, jnp.minimum(te, ne - 1))


def _dispatch_sc(x3, pos_a, p_pad):
    n, sl, _ = x3.shape
    chunk = pos_a.shape[1] * pos_a.shape[2]
    sub = pos_a.shape[2]
    mesh = plsc.VectorSubcoreMesh(core_axis_name="c", subcore_axis_name="s")

    @functools.partial(
        pl.kernel, mesh=mesh,
        out_type=jax.ShapeDtypeStruct((p_pad, sl, 128), jnp.int32),
        scratch_types=[
            pltpu.VMEM(pos_a.shape[1:], jnp.int32),
            pltpu.VMEM((sub, sl, 128), jnp.int32),
            pltpu.SemaphoreType.DMA,
        ],
    )
    def k(x_hbm, pos_hbm, out_hbm, idx_v, rows_v, sem):
        wid = lax.axis_index("s") * 2 + lax.axis_index("c")
        pltpu.sync_copy(pos_hbm.at[wid], idx_v)
        for l in range(pos_a.shape[1]):
            t0 = lax.rem(wid * chunk + l * sub, n)
            pltpu.sync_copy(x_hbm.at[pl.ds(t0, sub)], rows_v)
            pltpu.async_copy(rows_v, out_hbm.at[idx_v.at[l]], sem).wait()

    return k(x3, pos_a)


def _gather_sc(ys3, pos_b, p):
    sl = ys3.shape[1]
    chunk = pos_b.shape[1] * pos_b.shape[2]
    sub = pos_b.shape[2]
    mesh = plsc.VectorSubcoreMesh(core_axis_name="c", subcore_axis_name="s")

    @functools.partial(
        pl.kernel, mesh=mesh,
        out_type=jax.ShapeDtypeStruct((p, sl, 128), jnp.int32),
        scratch_types=[
            pltpu.VMEM(pos_b.shape[1:], jnp.int32),
            pltpu.VMEM((sub, sl, 128), jnp.int32),
            pltpu.SemaphoreType.DMA,
        ],
    )
    def k(ys_hbm, pos_hbm, out_hbm, idx_v, rows_v, sem):
        wid = lax.axis_index("s") * 2 + lax.axis_index("c")
        pltpu.sync_copy(pos_hbm.at[wid], idx_v)
        for l in range(pos_b.shape[1]):
            pltpu.async_copy(ys_hbm.at[idx_v.at[l]], rows_v, sem).wait()
            pltpu.sync_copy(rows_v, out_hbm.at[pl.ds(wid * chunk + l * sub, sub)])

    return k(ys3, pos_b)


def _gffn_body(meta_ref, x_ref, w1_ref, b1_ref, w2_ref, b2_ref, out_ref):
    i = pl.program_id(0)
    na = meta_ref[63]

    @pl.when(i < na)
    def _():
        h = jnp.dot(x_ref[...], w1_ref[0], preferred_element_type=jnp.float32) + b1_ref[0]
        h = jnp.maximum(h, 0.0).astype(jnp.bfloat16)
        y = jnp.dot(h, w2_ref[0], preferred_element_type=jnp.float32) + b2_ref[0]
        out_ref[...] = y.astype(jnp.bfloat16)


def _combine_body(y0_ref, y1_ref, w_ref, out_ref):
    out_ref[...] = (w_ref[:, 0:1] * y0_ref[...].astype(jnp.float32)
                    + w_ref[:, 1:2] * y1_ref[...].astype(jnp.float32))


def kernel(inputs, task_param, alpha, Wg_in, bg_in, Wg_task, bg_task, W1, b1, W2, b2):
    bsz, seq, dm = inputs.shape
    ne = Wg_in.shape[1]
    fd = W1.shape[2]
    n = bsz * seq
    m = 256
    p = n * 2
    p_pad = p + ne * m
    tf = p_pad // m

    xf = inputs.reshape(n, dm)
    tpf = task_param.reshape(n, dm)
    a = alpha.astype(jnp.float32)
    wgi = (1.0 - a) * Wg_in
    wgt = a * Wg_task
    bg = ((1.0 - a) * bg_in + a * bg_task).reshape(1, ne)

    mr = min(1024, n)
    nt = n // mr
    laux, w_nk, e_nk, r_nk, cnt16 = pl.pallas_call(
        functools.partial(_routing_body, nt, n),
        grid=(nt,),
        in_specs=[
            pl.BlockSpec((mr, dm), lambda t: (t, 0)),
            pl.BlockSpec((mr, dm), lambda t: (t, 0)),
            pl.BlockSpec((dm, ne), lambda t: (0, 0)),
            pl.BlockSpec((dm, ne), lambda t: (0, 0)),
            pl.BlockSpec((1, ne), lambda t: (0, 0)),
        ],
        out_specs=[
            pl.BlockSpec((1, 1), lambda t: (0, 0)),
            pl.BlockSpec((mr, 2), lambda t: (t, 0)),
            pl.BlockSpec((mr, 2), lambda t: (t, 0)),
            pl.BlockSpec((mr, 2), lambda t: (t, 0)),
            pl.BlockSpec((1, 2 * ne), lambda t: (0, 0)),
        ],
        out_shape=[
            jax.ShapeDtypeStruct((1, 1), jnp.float32),
            jax.ShapeDtypeStruct((n, 2), jnp.float32),
            jax.ShapeDtypeStruct((n, 2), jnp.int32),
            jax.ShapeDtypeStruct((n, 2), jnp.int32),
            jax.ShapeDtypeStruct((1, 2 * ne), jnp.int32),
        ],
        scratch_shapes=[pltpu.VMEM((2, ne), jnp.float32),
                        pltpu.VMEM((2, ne), jnp.int32)],
    )(xf, tpf, wgi, wgt, bg)

    pos_nk, meta = pl.pallas_call(
        functools.partial(_finalize_body, m, ne),
        grid_spec=pltpu.PrefetchScalarGridSpec(
            num_scalar_prefetch=1,
            grid=(nt,),
            in_specs=[
                pl.BlockSpec((mr, 2), lambda t, c: (t, 0)),
                pl.BlockSpec((mr, 2), lambda t, c: (t, 0)),
            ],
            out_specs=[
                pl.BlockSpec((mr, 2), lambda t, c: (t, 0)),
                pl.BlockSpec((1, 64), lambda t, c: (0, 0)),
            ],
        ),
        out_shape=[
            jax.ShapeDtypeStruct((n, 2), jnp.int32),
            jax.ShapeDtypeStruct((1, 64), jnp.int32),
        ],
    )(cnt16.reshape(2 * ne), e_nk, r_nk)

    posk = jnp.transpose(pos_nk, (1, 0)).reshape(p)
    pos_a = posk.reshape(NW, -1, 128)
    pos_b = posk.reshape(NW, -1, 64)

    xi3 = lax.bitcast_convert_type(
        xf.astype(jnp.bfloat16).reshape(n, dm // 2, 2), jnp.int32
    ).reshape(n, dm // 256, 128)
    xs3 = _dispatch_sc(xi3, pos_a, p_pad)

    ys = pl.pallas_call(
        _gffn_body,
        grid_spec=pltpu.PrefetchScalarGridSpec(
            num_scalar_prefetch=1,
            grid=(tf,),
            in_specs=[
                pl.BlockSpec((m, dm), lambda i, s: (i, 0)),
                pl.BlockSpec((1, dm, fd), lambda i, s: (s[i], 0, 0)),
                pl.BlockSpec((1, 1, fd), lambda i, s: (s[i], 0, 0)),
                pl.BlockSpec((1, fd, dm), lambda i, s: (s[i], 0, 0)),
                pl.BlockSpec((1, 1, dm), lambda i, s: (s[i], 0, 0)),
            ],
            out_specs=pl.BlockSpec((m, dm), lambda i, s: (i, 0)),
        ),
        out_shape=jax.ShapeDtypeStruct((p_pad, dm), jnp.bfloat16),
    )(meta.reshape(64),
      lax.bitcast_convert_type(xs3.reshape(p_pad, dm // 2), jnp.bfloat16
                               ).reshape(p_pad, dm),
      W1.astype(jnp.bfloat16), b1.reshape(ne, 1, fd),
      W2.astype(jnp.bfloat16), b2.reshape(ne, 1, dm))

    ysi = lax.bitcast_convert_type(ys.reshape(p_pad, dm // 2, 2), jnp.int32
                                   ).reshape(p_pad, dm // 256, 128)
    yg = _gather_sc(ysi, pos_b, p)
    y2 = lax.bitcast_convert_type(yg.reshape(p, dm // 2), jnp.bfloat16
                                  ).reshape(2, n, dm)

    res = pl.pallas_call(
        _combine_body,
        grid=(nt,),
        in_specs=[
            pl.BlockSpec((mr, dm), lambda t: (t, 0)),
            pl.BlockSpec((mr, dm), lambda t: (t, 0)),
            pl.BlockSpec((mr, 2), lambda t: (t, 0)),
        ],
        out_specs=pl.BlockSpec((mr, dm), lambda t: (t, 0)),
        out_shape=jax.ShapeDtypeStruct((n, dm), jnp.float32),
    )(y2[0], y2[1], w_nk)

    return res.reshape(bsz, seq, dm), laux[0, 0]


# speedup vs baseline: 3.0910x; 3.0910x over previous
import functools

import jax
import jax.numpy as jnp
from jax import lax
from jax.experimental import pallas as pl
from jax.experimental.pallas import tpu as pltpu
from jax.experimental.pallas import tpu_sc as plsc

NW = 32


def _routing_body(nt, ntok, x_ref, tp_ref, wgi_ref, wgt_ref, bg_ref,
                  laux_ref, w_ref, e_ref, r_ref, cnt16_ref, acc_ref, cnt_ref):
    t = pl.program_id(0)
    logits = (jnp.dot(x_ref[...], wgi_ref[...], preferred_element_type=jnp.float32)
              + jnp.dot(tp_ref[...], wgt_ref[...], preferred_element_type=jnp.float32)
              + bg_ref[...])
    mr, ne = logits.shape
    lane = lax.broadcasted_iota(jnp.int32, logits.shape, 1)
    m1 = jnp.max(logits, axis=1, keepdims=True)
    i1 = jnp.min(jnp.where(logits == m1, lane, ne), axis=1, keepdims=True)
    masked = jnp.where(lane == i1, -jnp.inf, logits)
    m2 = jnp.max(masked, axis=1, keepdims=True)
    i2 = jnp.min(jnp.where(masked == m2, lane, ne), axis=1, keepdims=True)
    s = jnp.exp(m2 - m1)
    w1 = 1.0 / (1.0 + s)
    w2 = s / (1.0 + s)
    sel1 = lane == i1
    sel2 = lane == i2

    @pl.when(t == 0)
    def _():
        acc_ref[...] = jnp.zeros_like(acc_ref)
        cnt_ref[...] = jnp.zeros_like(cnt_ref)

    o0 = sel1.astype(jnp.float32)
    o1 = sel2.astype(jnp.float32)
    row = lax.broadcasted_iota(jnp.int32, (mr, mr), 0)
    col = lax.broadcasted_iota(jnp.int32, (mr, mr), 1)
    tril = (col < row).astype(jnp.float32)
    rk0 = jnp.dot(tril, o0, preferred_element_type=jnp.float32)
    rk1 = jnp.dot(tril, o1, preferred_element_type=jnp.float32)
    base0 = cnt_ref[0:1, :].astype(jnp.float32)
    base1 = cnt_ref[1:2, :].astype(jnp.float32)
    r0 = jnp.sum(jnp.where(sel1, rk0 + base0, 0.0), axis=1, keepdims=True)
    r1 = jnp.sum(jnp.where(sel2, rk1 + base1, 0.0), axis=1, keepdims=True)
    r_ref[...] = jnp.concatenate([r0, r1], axis=1).astype(jnp.int32)
    e_ref[...] = jnp.concatenate([i1, i2], axis=1)
    w_ref[...] = jnp.concatenate([w1, w2], axis=1)
    cnt_ref[0:1, :] += jnp.sum(o0, axis=0, keepdims=True).astype(jnp.int32)
    cnt_ref[1:2, :] += jnp.sum(o1, axis=0, keepdims=True).astype(jnp.int32)

    p = jnp.exp(logits - m1)
    p = p / jnp.sum(p, axis=1, keepdims=True)
    acc_ref[0:1, :] += jnp.sum(p, axis=0, keepdims=True)
    acc_ref[1:2, :] += jnp.sum(o0 + o1, axis=0, keepdims=True)

    @pl.when(t == nt - 1)
    def _():
        laux_ref[...] = (jnp.sum(acc_ref[0:1, :] * acc_ref[1:2, :])
                         / (ntok * ntok)).reshape(1, 1)
        cnt16_ref[...] = jnp.concatenate([cnt_ref[0:1, :], cnt_ref[1:2, :]], axis=1)


def _finalize_body(m, ne, cnt_ref, e_ref, r_ref, pos_ref, meta_ref):
    t = pl.program_id(0)
    e_nk = e_ref[...]
    kk = lax.broadcasted_iota(jnp.int32, e_nk.shape, 1)
    lane64 = lax.broadcasted_iota(jnp.int32, (1, 64), 1)
    add = jnp.zeros(e_nk.shape, jnp.int32)
    te = jnp.zeros((1, 64), jnp.int32)
    po = jnp.int32(0)
    btl = jnp.int32(0)
    for e in range(ne):
        c0 = cnt_ref[e]
        c1 = cnt_ref[ne + e]
        pc = ((c0 + c1 + m - 1) // m) * m
        add = add + jnp.where(e_nk == e, po + jnp.where(kk == 1, c0, 0), 0)
        po = po + pc
        btl = btl + pc // m
        te = te + (lane64 >= btl).astype(jnp.int32)
    pos_ref[...] = r_ref[...] + add
    na = btl

    @pl.when(t == 0)
    def _():
        meta_ref[...] = jnp.where(lane64 == 63, na, jnp.minimum(te, ne - 1))


def _dispatch_sc(x, pos_a, p_pad):
    n, dm = x.shape
    chunk = pos_a.shape[1] * pos_a.shape[2]
    sub = pos_a.shape[2]
    mesh = plsc.VectorSubcoreMesh(core_axis_name="c", subcore_axis_name="s")

    @functools.partial(
        pl.kernel, mesh=mesh,
        out_type=jax.ShapeDtypeStruct((p_pad, dm), jnp.float32),
        scratch_types=[
            pltpu.VMEM(pos_a.shape[1:], jnp.int32),
            pltpu.VMEM((sub, dm), jnp.float32),
            pltpu.SemaphoreType.DMA,
        ],
    )
    def k(x_hbm, pos_hbm, out_hbm, idx_v, rows_v, sem):
        wid = lax.axis_index("s") * 2 + lax.axis_index("c")
        pltpu.sync_copy(pos_hbm.at[wid], idx_v)
        for l in range(pos_a.shape[1]):
            t0 = lax.rem(wid * chunk + l * sub, n)
            pltpu.sync_copy(x_hbm.at[pl.ds(t0, sub)], rows_v)
            pltpu.async_copy(rows_v, out_hbm.at[idx_v.at[l]], sem).wait()

    return k(x, pos_a)


def _gather_sc(ys, pos_b, p):
    dm = ys.shape[1]
    chunk = pos_b.shape[1] * pos_b.shape[2]
    sub = pos_b.shape[2]
    mesh = plsc.VectorSubcoreMesh(core_axis_name="c", subcore_axis_name="s")

    @functools.partial(
        pl.kernel, mesh=mesh,
        out_type=jax.ShapeDtypeStruct((p, dm), jnp.float32),
        scratch_types=[
            pltpu.VMEM(pos_b.shape[1:], jnp.int32),
            pltpu.VMEM((sub, dm), jnp.float32),
            pltpu.SemaphoreType.DMA,
        ],
    )
    def k(ys_hbm, pos_hbm, out_hbm, idx_v, rows_v, sem):
        wid = lax.axis_index("s") * 2 + lax.axis_index("c")
        pltpu.sync_copy(pos_hbm.at[wid], idx_v)
        for l in range(pos_b.shape[1]):
            pltpu.async_copy(ys_hbm.at[idx_v.at[l]], rows_v, sem).wait()
            pltpu.sync_copy(rows_v, out_hbm.at[pl.ds(wid * chunk + l * sub, sub)])

    return k(ys, pos_b)


def _gffn_body(meta_ref, x_ref, w1_ref, b1_ref, w2_ref, b2_ref, out_ref):
    i = pl.program_id(0)
    na = meta_ref[63]

    @pl.when(i < na)
    def _():
        xb = x_ref[...].astype(jnp.bfloat16)
        h = jnp.dot(xb, w1_ref[0], preferred_element_type=jnp.float32) + b1_ref[0]
        h = jnp.maximum(h, 0.0).astype(jnp.bfloat16)
        y = jnp.dot(h, w2_ref[0], preferred_element_type=jnp.float32) + b2_ref[0]
        out_ref[...] = y


def _combine_body(y0_ref, y1_ref, w_ref, out_ref):
    out_ref[...] = (w_ref[:, 0:1] * y0_ref[...]
                    + w_ref[:, 1:2] * y1_ref[...])


def kernel(inputs, task_param, alpha, Wg_in, bg_in, Wg_task, bg_task, W1, b1, W2, b2):
    bsz, seq, dm = inputs.shape
    ne = Wg_in.shape[1]
    fd = W1.shape[2]
    n = bsz * seq
    m = 256
    p = n * 2
    p_pad = p + ne * m
    tf = p_pad // m

    xf = inputs.reshape(n, dm)
    tpf = task_param.reshape(n, dm)
    a = alpha.astype(jnp.float32)
    wgi = (1.0 - a) * Wg_in
    wgt = a * Wg_task
    bg = ((1.0 - a) * bg_in + a * bg_task).reshape(1, ne)

    mr = min(1024, n)
    nt = n // mr
    laux, w_nk, e_nk, r_nk, cnt16 = pl.pallas_call(
        functools.partial(_routing_body, nt, n),
        grid=(nt,),
        in_specs=[
            pl.BlockSpec((mr, dm), lambda t: (t, 0)),
            pl.BlockSpec((mr, dm), lambda t: (t, 0)),
            pl.BlockSpec((dm, ne), lambda t: (0, 0)),
            pl.BlockSpec((dm, ne), lambda t: (0, 0)),
            pl.BlockSpec((1, ne), lambda t: (0, 0)),
        ],
        out_specs=[
            pl.BlockSpec((1, 1), lambda t: (0, 0)),
            pl.BlockSpec((mr, 2), lambda t: (t, 0)),
            pl.BlockSpec((mr, 2), lambda t: (t, 0)),
            pl.BlockSpec((mr, 2), lambda t: (t, 0)),
            pl.BlockSpec((1, 2 * ne), lambda t: (0, 0)),
        ],
        out_shape=[
            jax.ShapeDtypeStruct((1, 1), jnp.float32),
            jax.ShapeDtypeStruct((n, 2), jnp.float32),
            jax.ShapeDtypeStruct((n, 2), jnp.int32),
            jax.ShapeDtypeStruct((n, 2), jnp.int32),
            jax.ShapeDtypeStruct((1, 2 * ne), jnp.int32),
        ],
        scratch_shapes=[pltpu.VMEM((2, ne), jnp.float32),
                        pltpu.VMEM((2, ne), jnp.int32)],
    )(xf, tpf, wgi, wgt, bg)

    pos_nk, meta = pl.pallas_call(
        functools.partial(_finalize_body, m, ne),
        grid_spec=pltpu.PrefetchScalarGridSpec(
            num_scalar_prefetch=1,
            grid=(nt,),
            in_specs=[
                pl.BlockSpec((mr, 2), lambda t, c: (t, 0)),
                pl.BlockSpec((mr, 2), lambda t, c: (t, 0)),
            ],
            out_specs=[
                pl.BlockSpec((mr, 2), lambda t, c: (t, 0)),
                pl.BlockSpec((1, 64), lambda t, c: (0, 0)),
            ],
        ),
        out_shape=[
            jax.ShapeDtypeStruct((n, 2), jnp.int32),
            jax.ShapeDtypeStruct((1, 64), jnp.int32),
        ],
    )(cnt16.reshape(2 * ne), e_nk, r_nk)

    posk = jnp.transpose(pos_nk, (1, 0)).reshape(p)
    pos_a = posk.reshape(NW, -1, 64)

    xs = _dispatch_sc(xf, pos_a, p_pad)

    ys = pl.pallas_call(
        _gffn_body,
        grid_spec=pltpu.PrefetchScalarGridSpec(
            num_scalar_prefetch=1,
            grid=(tf,),
            in_specs=[
                pl.BlockSpec((m, dm), lambda i, s: (i, 0)),
                pl.BlockSpec((1, dm, fd), lambda i, s: (s[i], 0, 0)),
                pl.BlockSpec((1, 1, fd), lambda i, s: (s[i], 0, 0)),
                pl.BlockSpec((1, fd, dm), lambda i, s: (s[i], 0, 0)),
                pl.BlockSpec((1, 1, dm), lambda i, s: (s[i], 0, 0)),
            ],
            out_specs=pl.BlockSpec((m, dm), lambda i, s: (i, 0)),
        ),
        out_shape=jax.ShapeDtypeStruct((p_pad, dm), jnp.float32),
    )(meta.reshape(64), xs,
      W1.astype(jnp.bfloat16), b1.reshape(ne, 1, fd),
      W2.astype(jnp.bfloat16), b2.reshape(ne, 1, dm))

    yg = _gather_sc(ys, pos_a, p)
    y2 = yg.reshape(2, n, dm)

    res = pl.pallas_call(
        _combine_body,
        grid=(nt,),
        in_specs=[
            pl.BlockSpec((mr, dm), lambda t: (t, 0)),
            pl.BlockSpec((mr, dm), lambda t: (t, 0)),
            pl.BlockSpec((mr, 2), lambda t: (t, 0)),
        ],
        out_specs=pl.BlockSpec((mr, dm), lambda t: (t, 0)),
        out_shape=jax.ShapeDtypeStruct((n, dm), jnp.float32),
    )(y2[0], y2[1], w_nk)

    return res.reshape(bsz, seq, dm), laux[0, 0]


# speedup vs baseline: 3.1572x; 1.0214x over previous
import functools

import jax
import jax.numpy as jnp
from jax import lax
from jax.experimental import pallas as pl
from jax.experimental.pallas import tpu as pltpu
from jax.experimental.pallas import tpu_sc as plsc

NW = 32


def _routing_body(nt, ntok, x_ref, tp_ref, wgi_ref, wgt_ref, bg_ref,
                  laux_ref, w_ref, e_ref, r_ref, cnt16_ref, acc_ref, cnt_ref):
    t = pl.program_id(0)
    logits = (jnp.dot(x_ref[...], wgi_ref[...], preferred_element_type=jnp.float32)
              + jnp.dot(tp_ref[...], wgt_ref[...], preferred_element_type=jnp.float32)
              + bg_ref[...])
    mr, ne = logits.shape
    lane = lax.broadcasted_iota(jnp.int32, logits.shape, 1)
    m1 = jnp.max(logits, axis=1, keepdims=True)
    i1 = jnp.min(jnp.where(logits == m1, lane, ne), axis=1, keepdims=True)
    masked = jnp.where(lane == i1, -jnp.inf, logits)
    m2 = jnp.max(masked, axis=1, keepdims=True)
    i2 = jnp.min(jnp.where(masked == m2, lane, ne), axis=1, keepdims=True)
    s = jnp.exp(m2 - m1)
    w1 = 1.0 / (1.0 + s)
    w2 = s / (1.0 + s)
    sel1 = lane == i1
    sel2 = lane == i2

    @pl.when(t == 0)
    def _():
        acc_ref[...] = jnp.zeros_like(acc_ref)
        cnt_ref[...] = jnp.zeros_like(cnt_ref)

    o0 = sel1.astype(jnp.float32)
    o1 = sel2.astype(jnp.float32)
    row = lax.broadcasted_iota(jnp.int32, (mr, mr), 0)
    col = lax.broadcasted_iota(jnp.int32, (mr, mr), 1)
    tril = (col < row).astype(jnp.float32)
    rk0 = jnp.dot(tril, o0, preferred_element_type=jnp.float32)
    rk1 = jnp.dot(tril, o1, preferred_element_type=jnp.float32)
    base0 = cnt_ref[0:1, :].astype(jnp.float32)
    base1 = cnt_ref[1:2, :].astype(jnp.float32)
    r0 = jnp.sum(jnp.where(sel1, rk0 + base0, 0.0), axis=1, keepdims=True)
    r1 = jnp.sum(jnp.where(sel2, rk1 + base1, 0.0), axis=1, keepdims=True)
    r_ref[...] = jnp.concatenate([r0, r1], axis=1).astype(jnp.int32)
    e_ref[...] = jnp.concatenate([i1, i2], axis=1)
    w_ref[...] = jnp.concatenate([w1, w2], axis=1)
    cnt_ref[0:1, :] += jnp.sum(o0, axis=0, keepdims=True).astype(jnp.int32)
    cnt_ref[1:2, :] += jnp.sum(o1, axis=0, keepdims=True).astype(jnp.int32)

    p = jnp.exp(logits - m1)
    p = p / jnp.sum(p, axis=1, keepdims=True)
    acc_ref[0:1, :] += jnp.sum(p, axis=0, keepdims=True)
    acc_ref[1:2, :] += jnp.sum(o0 + o1, axis=0, keepdims=True)

    @pl.when(t == nt - 1)
    def _():
        laux_ref[...] = (jnp.sum(acc_ref[0:1, :] * acc_ref[1:2, :])
                         / (ntok * ntok)).reshape(1, 1)
        cnt16_ref[...] = jnp.concatenate([cnt_ref[0:1, :], cnt_ref[1:2, :]], axis=1)


def _finalize_body(m, ne, cnt_ref, e_ref, r_ref, pos_ref, meta_ref):
    t = pl.program_id(0)
    e_nk = e_ref[...]
    kk = lax.broadcasted_iota(jnp.int32, e_nk.shape, 1)
    lane64 = lax.broadcasted_iota(jnp.int32, (1, 64), 1)
    add = jnp.zeros(e_nk.shape, jnp.int32)
    te = jnp.zeros((1, 64), jnp.int32)
    po = jnp.int32(0)
    btl = jnp.int32(0)
    for e in range(ne):
        c0 = cnt_ref[e]
        c1 = cnt_ref[ne + e]
        pc = ((c0 + c1 + m - 1) // m) * m
        add = add + jnp.where(e_nk == e, po + jnp.where(kk == 1, c0, 0), 0)
        po = po + pc
        btl = btl + pc // m
        te = te + (lane64 >= btl).astype(jnp.int32)
    pos_ref[...] = r_ref[...] + add
    na = btl

    @pl.when(t == 0)
    def _():
        meta_ref[...] = jnp.where(lane64 == 63, na, jnp.minimum(te, ne - 1))


def _dispatch_sc(x, pos_a, p_pad):
    n, dm = x.shape
    nk, nl, sub = pos_a.shape[1:]
    mesh = plsc.VectorSubcoreMesh(core_axis_name="c", subcore_axis_name="s")

    @functools.partial(
        pl.kernel, mesh=mesh,
        out_type=jax.ShapeDtypeStruct((p_pad, dm), jnp.float32),
        scratch_types=[
            pltpu.VMEM(pos_a.shape[1:], jnp.int32),
            pltpu.VMEM((sub, dm), jnp.float32),
            pltpu.SemaphoreType.DMA,
        ],
    )
    def k(x_hbm, pos_hbm, out_hbm, idx_v, rows_v, sem):
        wid = lax.axis_index("s") * 2 + lax.axis_index("c")
        pltpu.sync_copy(pos_hbm.at[wid], idx_v)
        for l in range(nl):
            t0 = wid * nl * sub + l * sub
            pltpu.sync_copy(x_hbm.at[pl.ds(t0, sub)], rows_v)
            cps = [pltpu.async_copy(rows_v, out_hbm.at[idx_v.at[kk, l]], sem)
                   for kk in range(nk)]
            for c in cps:
                c.wait()

    return k(x, pos_a)


def _gather_sc(ys, pos_b, p):
    dm = ys.shape[1]
    chunk = pos_b.shape[1] * pos_b.shape[2]
    sub = pos_b.shape[2]
    mesh = plsc.VectorSubcoreMesh(core_axis_name="c", subcore_axis_name="s")

    @functools.partial(
        pl.kernel, mesh=mesh,
        out_type=jax.ShapeDtypeStruct((p, dm), jnp.float32),
        scratch_types=[
            pltpu.VMEM(pos_b.shape[1:], jnp.int32),
            pltpu.VMEM((sub, dm), jnp.float32),
            pltpu.SemaphoreType.DMA,
        ],
    )
    def k(ys_hbm, pos_hbm, out_hbm, idx_v, rows_v, sem):
        wid = lax.axis_index("s") * 2 + lax.axis_index("c")
        pltpu.sync_copy(pos_hbm.at[wid], idx_v)
        for l in range(pos_b.shape[1]):
            pltpu.async_copy(ys_hbm.at[idx_v.at[l]], rows_v, sem).wait()
            pltpu.sync_copy(rows_v, out_hbm.at[pl.ds(wid * chunk + l * sub, sub)])

    return k(ys, pos_b)


def _gffn_body(meta_ref, x_ref, w1_ref, b1_ref, w2_ref, b2_ref, out_ref):
    i = pl.program_id(0)
    na = meta_ref[63]

    @pl.when(i < na)
    def _():
        xb = x_ref[...].astype(jnp.bfloat16)
        h = jnp.dot(xb, w1_ref[0], preferred_element_type=jnp.float32) + b1_ref[0]
        h = jnp.maximum(h, 0.0).astype(jnp.bfloat16)
        y = jnp.dot(h, w2_ref[0], preferred_element_type=jnp.float32) + b2_ref[0]
        out_ref[...] = y


def _combine_body(y0_ref, y1_ref, w_ref, out_ref):
    out_ref[...] = (w_ref[:, 0:1] * y0_ref[...]
                    + w_ref[:, 1:2] * y1_ref[...])


def kernel(inputs, task_param, alpha, Wg_in, bg_in, Wg_task, bg_task, W1, b1, W2, b2):
    bsz, seq, dm = inputs.shape
    ne = Wg_in.shape[1]
    fd = W1.shape[2]
    n = bsz * seq
    m = 256
    p = n * 2
    p_pad = p + ne * m
    tf = p_pad // m

    xf = inputs.reshape(n, dm)
    tpf = task_param.reshape(n, dm)
    a = alpha.astype(jnp.float32)
    wgi = (1.0 - a) * Wg_in
    wgt = a * Wg_task
    bg = ((1.0 - a) * bg_in + a * bg_task).reshape(1, ne)

    mr = min(1024, n)
    nt = n // mr
    laux, w_nk, e_nk, r_nk, cnt16 = pl.pallas_call(
        functools.partial(_routing_body, nt, n),
        grid=(nt,),
        in_specs=[
            pl.BlockSpec((mr, dm), lambda t: (t, 0)),
            pl.BlockSpec((mr, dm), lambda t: (t, 0)),
            pl.BlockSpec((dm, ne), lambda t: (0, 0)),
            pl.BlockSpec((dm, ne), lambda t: (0, 0)),
            pl.BlockSpec((1, ne), lambda t: (0, 0)),
        ],
        out_specs=[
            pl.BlockSpec((1, 1), lambda t: (0, 0)),
            pl.BlockSpec((mr, 2), lambda t: (t, 0)),
            pl.BlockSpec((mr, 2), lambda t: (t, 0)),
            pl.BlockSpec((mr, 2), lambda t: (t, 0)),
            pl.BlockSpec((1, 2 * ne), lambda t: (0, 0)),
        ],
        out_shape=[
            jax.ShapeDtypeStruct((1, 1), jnp.float32),
            jax.ShapeDtypeStruct((n, 2), jnp.float32),
            jax.ShapeDtypeStruct((n, 2), jnp.int32),
            jax.ShapeDtypeStruct((n, 2), jnp.int32),
            jax.ShapeDtypeStruct((1, 2 * ne), jnp.int32),
        ],
        scratch_shapes=[pltpu.VMEM((2, ne), jnp.float32),
                        pltpu.VMEM((2, ne), jnp.int32)],
    )(xf, tpf, wgi, wgt, bg)

    pos_nk, meta = pl.pallas_call(
        functools.partial(_finalize_body, m, ne),
        grid_spec=pltpu.PrefetchScalarGridSpec(
            num_scalar_prefetch=1,
            grid=(nt,),
            in_specs=[
                pl.BlockSpec((mr, 2), lambda t, c: (t, 0)),
                pl.BlockSpec((mr, 2), lambda t, c: (t, 0)),
            ],
            out_specs=[
                pl.BlockSpec((mr, 2), lambda t, c: (t, 0)),
                pl.BlockSpec((1, 64), lambda t, c: (0, 0)),
            ],
        ),
        out_shape=[
            jax.ShapeDtypeStruct((n, 2), jnp.int32),
            jax.ShapeDtypeStruct((1, 64), jnp.int32),
        ],
    )(cnt16.reshape(2 * ne), e_nk, r_nk)

    posk = jnp.transpose(pos_nk, (1, 0)).reshape(p)
    pos_b = posk.reshape(NW, -1, 64)
    pos_a = jnp.transpose(posk.reshape(2, NW, -1, 64), (1, 0, 2, 3))

    xs = _dispatch_sc(xf, pos_a, p_pad)

    ys = pl.pallas_call(
        _gffn_body,
        grid_spec=pltpu.PrefetchScalarGridSpec(
            num_scalar_prefetch=1,
            grid=(tf,),
            in_specs=[
                pl.BlockSpec((m, dm), lambda i, s: (i, 0)),
                pl.BlockSpec((1, dm, fd), lambda i, s: (s[i], 0, 0)),
                pl.BlockSpec((1, 1, fd), lambda i, s: (s[i], 0, 0)),
                pl.BlockSpec((1, fd, dm), lambda i, s: (s[i], 0, 0)),
                pl.BlockSpec((1, 1, dm), lambda i, s: (s[i], 0, 0)),
            ],
            out_specs=pl.BlockSpec((m, dm), lambda i, s: (i, 0)),
        ),
        out_shape=jax.ShapeDtypeStruct((p_pad, dm), jnp.float32),
    )(meta.reshape(64), xs,
      W1.astype(jnp.bfloat16), b1.reshape(ne, 1, fd),
      W2.astype(jnp.bfloat16), b2.reshape(ne, 1, dm))

    yg = _gather_sc(ys, pos_b, p)
    y2 = yg.reshape(2, n, dm)

    res = pl.pallas_call(
        _combine_body,
        grid=(nt,),
        in_specs=[
            pl.BlockSpec((mr, dm), lambda t: (t, 0)),
            pl.BlockSpec((mr, dm), lambda t: (t, 0)),
            pl.BlockSpec((mr, 2), lambda t: (t, 0)),
        ],
        out_specs=pl.BlockSpec((mr, dm), lambda t: (t, 0)),
        out_shape=jax.ShapeDtypeStruct((n, dm), jnp.float32),
    )(y2[0], y2[1], w_nk)

    return res.reshape(bsz, seq, dm), laux[0, 0]


# speedup vs baseline: 3.6067x; 1.1424x over previous
import functools

import jax
import jax.numpy as jnp
from jax import lax
from jax.experimental import pallas as pl
from jax.experimental.pallas import tpu as pltpu
from jax.experimental.pallas import tpu_sc as plsc

NW = 32


def _routing_body(nt, ntok, x_ref, tp_ref, wgi_ref, wgt_ref, bg_ref,
                  laux_ref, w_ref, e_ref, r_ref, cnt16_ref, acc_ref, cnt_ref):
    t = pl.program_id(0)
    logits = (jnp.dot(x_ref[...], wgi_ref[...], preferred_element_type=jnp.float32)
              + jnp.dot(tp_ref[...], wgt_ref[...], preferred_element_type=jnp.float32)
              + bg_ref[...])
    mr, ne = logits.shape
    lane = lax.broadcasted_iota(jnp.int32, logits.shape, 1)
    m1 = jnp.max(logits, axis=1, keepdims=True)
    i1 = jnp.min(jnp.where(logits == m1, lane, ne), axis=1, keepdims=True)
    masked = jnp.where(lane == i1, -jnp.inf, logits)
    m2 = jnp.max(masked, axis=1, keepdims=True)
    i2 = jnp.min(jnp.where(masked == m2, lane, ne), axis=1, keepdims=True)
    s = jnp.exp(m2 - m1)
    w1 = 1.0 / (1.0 + s)
    w2 = s / (1.0 + s)
    sel1 = lane == i1
    sel2 = lane == i2

    @pl.when(t == 0)
    def _():
        acc_ref[...] = jnp.zeros_like(acc_ref)
        cnt_ref[...] = jnp.zeros_like(cnt_ref)

    o0 = sel1.astype(jnp.float32)
    o1 = sel2.astype(jnp.float32)
    row = lax.broadcasted_iota(jnp.int32, (mr, mr), 0)
    col = lax.broadcasted_iota(jnp.int32, (mr, mr), 1)
    tril = (col < row).astype(jnp.float32)
    rk0 = jnp.dot(tril, o0, preferred_element_type=jnp.float32)
    rk1 = jnp.dot(tril, o1, preferred_element_type=jnp.float32)
    base0 = cnt_ref[0:1, :].astype(jnp.float32)
    base1 = cnt_ref[1:2, :].astype(jnp.float32)
    r0 = jnp.sum(jnp.where(sel1, rk0 + base0, 0.0), axis=1, keepdims=True)
    r1 = jnp.sum(jnp.where(sel2, rk1 + base1, 0.0), axis=1, keepdims=True)
    r_ref[...] = jnp.concatenate([r0, r1], axis=1).astype(jnp.int32)
    e_ref[...] = jnp.concatenate([i1, i2], axis=1)
    w_ref[...] = jnp.concatenate([w1, w2], axis=1)
    cnt_ref[0:1, :] += jnp.sum(o0, axis=0, keepdims=True).astype(jnp.int32)
    cnt_ref[1:2, :] += jnp.sum(o1, axis=0, keepdims=True).astype(jnp.int32)

    p = jnp.exp(logits - m1)
    p = p / jnp.sum(p, axis=1, keepdims=True)
    acc_ref[0:1, :] += jnp.sum(p, axis=0, keepdims=True)
    acc_ref[1:2, :] += jnp.sum(o0 + o1, axis=0, keepdims=True)

    @pl.when(t == nt - 1)
    def _():
        laux_ref[...] = (jnp.sum(acc_ref[0:1, :] * acc_ref[1:2, :])
                         / (ntok * ntok)).reshape(1, 1)
        cnt16_ref[...] = jnp.concatenate([cnt_ref[0:1, :], cnt_ref[1:2, :]], axis=1)


def _finalize_body(m, ne, cnt_ref, e_ref, r_ref, pos_ref, meta_ref):
    t = pl.program_id(0)
    e_nk = e_ref[...]
    kk = lax.broadcasted_iota(jnp.int32, e_nk.shape, 1)
    lane64 = lax.broadcasted_iota(jnp.int32, (1, 64), 1)
    add = jnp.zeros(e_nk.shape, jnp.int32)
    te = jnp.zeros((1, 64), jnp.int32)
    po = jnp.int32(0)
    btl = jnp.int32(0)
    for e in range(ne):
        c0 = cnt_ref[e]
        c1 = cnt_ref[ne + e]
        pc = ((c0 + c1 + m - 1) // m) * m
        add = add + jnp.where(e_nk == e, po + jnp.where(kk == 1, c0, 0), 0)
        po = po + pc
        btl = btl + pc // m
        te = te + (lane64 >= btl).astype(jnp.int32)
    pos_ref[...] = r_ref[...] + add
    na = btl

    @pl.when(t == 0)
    def _():
        meta_ref[...] = jnp.where(lane64 == 63, na, jnp.minimum(te, ne - 1))


def _dispatch_sc(x, pos_a, p_pad):
    n, dm = x.shape
    nk, nl, sub = pos_a.shape[1:]
    mesh = plsc.VectorSubcoreMesh(core_axis_name="c", subcore_axis_name="s")

    @functools.partial(
        pl.kernel, mesh=mesh,
        out_type=jax.ShapeDtypeStruct((p_pad, dm), jnp.float32),
        scratch_types=[
            pltpu.VMEM(pos_a.shape[1:], jnp.int32),
            pltpu.VMEM((sub, dm), jnp.float32),
            pltpu.SemaphoreType.DMA,
        ],
    )
    def k(x_hbm, pos_hbm, out_hbm, idx_v, rows_v, sem):
        wid = lax.axis_index("s") * 2 + lax.axis_index("c")
        pltpu.sync_copy(pos_hbm.at[wid], idx_v)
        for l in range(nl):
            t0 = wid * nl * sub + l * sub
            pltpu.sync_copy(x_hbm.at[pl.ds(t0, sub)], rows_v)
            cps = [pltpu.async_copy(rows_v, out_hbm.at[idx_v.at[kk, l]], sem)
                   for kk in range(nk)]
            for c in cps:
                c.wait()

    return k(x, pos_a)


def _gather_sc(ys, pos_b, p):
    dm = ys.shape[1]
    chunk = pos_b.shape[1] * pos_b.shape[2]
    sub = pos_b.shape[2]
    mesh = plsc.VectorSubcoreMesh(core_axis_name="c", subcore_axis_name="s")

    @functools.partial(
        pl.kernel, mesh=mesh,
        out_type=jax.ShapeDtypeStruct((p, dm), jnp.float32),
        scratch_types=[
            pltpu.VMEM(pos_b.shape[1:], jnp.int32),
            pltpu.VMEM((sub, dm), jnp.float32),
            pltpu.SemaphoreType.DMA,
        ],
    )
    def k(ys_hbm, pos_hbm, out_hbm, idx_v, rows_v, sem):
        wid = lax.axis_index("s") * 2 + lax.axis_index("c")
        pltpu.sync_copy(pos_hbm.at[wid], idx_v)
        for l in range(pos_b.shape[1]):
            pltpu.async_copy(ys_hbm.at[idx_v.at[l]], rows_v, sem).wait()
            pltpu.sync_copy(rows_v, out_hbm.at[pl.ds(wid * chunk + l * sub, sub)])

    return k(ys, pos_b)


def _gffn_body(meta_ref, x_ref, w1_ref, b1_ref, w2_ref, b2_ref, out_ref):
    i = pl.program_id(0)
    na = meta_ref[63]

    @pl.when(i < na)
    def _():
        xb = x_ref[...].astype(jnp.bfloat16)
        w1 = w1_ref[0].astype(jnp.bfloat16)
        h = jnp.dot(xb, w1, preferred_element_type=jnp.float32) + b1_ref[0]
        h = jnp.maximum(h, 0.0).astype(jnp.bfloat16)
        w2 = w2_ref[0].astype(jnp.bfloat16)
        y = jnp.dot(h, w2, preferred_element_type=jnp.float32) + b2_ref[0]
        out_ref[...] = y


def _combine_body(y0_ref, y1_ref, w_ref, out_ref):
    out_ref[...] = (w_ref[:, 0:1] * y0_ref[...]
                    + w_ref[:, 1:2] * y1_ref[...])


def kernel(inputs, task_param, alpha, Wg_in, bg_in, Wg_task, bg_task, W1, b1, W2, b2):
    bsz, seq, dm = inputs.shape
    ne = Wg_in.shape[1]
    fd = W1.shape[2]
    n = bsz * seq
    m = 256
    p = n * 2
    p_pad = p + ne * m
    tf = p_pad // m

    xf = inputs.reshape(n, dm)
    tpf = task_param.reshape(n, dm)
    a = alpha.astype(jnp.float32)
    wgi = (1.0 - a) * Wg_in
    wgt = a * Wg_task
    bg = ((1.0 - a) * bg_in + a * bg_task).reshape(1, ne)

    mr = min(1024, n)
    nt = n // mr
    laux, w_nk, e_nk, r_nk, cnt16 = pl.pallas_call(
        functools.partial(_routing_body, nt, n),
        grid=(nt,),
        in_specs=[
            pl.BlockSpec((mr, dm), lambda t: (t, 0)),
            pl.BlockSpec((mr, dm), lambda t: (t, 0)),
            pl.BlockSpec((dm, ne), lambda t: (0, 0)),
            pl.BlockSpec((dm, ne), lambda t: (0, 0)),
            pl.BlockSpec((1, ne), lambda t: (0, 0)),
        ],
        out_specs=[
            pl.BlockSpec((1, 1), lambda t: (0, 0)),
            pl.BlockSpec((mr, 2), lambda t: (t, 0)),
            pl.BlockSpec((mr, 2), lambda t: (t, 0)),
            pl.BlockSpec((mr, 2), lambda t: (t, 0)),
            pl.BlockSpec((1, 2 * ne), lambda t: (0, 0)),
        ],
        out_shape=[
            jax.ShapeDtypeStruct((1, 1), jnp.float32),
            jax.ShapeDtypeStruct((n, 2), jnp.float32),
            jax.ShapeDtypeStruct((n, 2), jnp.int32),
            jax.ShapeDtypeStruct((n, 2), jnp.int32),
            jax.ShapeDtypeStruct((1, 2 * ne), jnp.int32),
        ],
        scratch_shapes=[pltpu.VMEM((2, ne), jnp.float32),
                        pltpu.VMEM((2, ne), jnp.int32)],
    )(xf, tpf, wgi, wgt, bg)

    pos_nk, meta = pl.pallas_call(
        functools.partial(_finalize_body, m, ne),
        grid_spec=pltpu.PrefetchScalarGridSpec(
            num_scalar_prefetch=1,
            grid=(nt,),
            in_specs=[
                pl.BlockSpec((mr, 2), lambda t, c: (t, 0)),
                pl.BlockSpec((mr, 2), lambda t, c: (t, 0)),
            ],
            out_specs=[
                pl.BlockSpec((mr, 2), lambda t, c: (t, 0)),
                pl.BlockSpec((1, 64), lambda t, c: (0, 0)),
            ],
        ),
        out_shape=[
            jax.ShapeDtypeStruct((n, 2), jnp.int32),
            jax.ShapeDtypeStruct((1, 64), jnp.int32),
        ],
    )(cnt16.reshape(2 * ne), e_nk, r_nk)

    posk = jnp.transpose(pos_nk, (1, 0)).reshape(p)
    pos_b = posk.reshape(NW, -1, 64)
    pos_a = jnp.transpose(posk.reshape(2, NW, -1, 64), (1, 0, 2, 3))

    xs = _dispatch_sc(xf, pos_a, p_pad)

    ys = pl.pallas_call(
        _gffn_body,
        grid_spec=pltpu.PrefetchScalarGridSpec(
            num_scalar_prefetch=1,
            grid=(tf,),
            in_specs=[
                pl.BlockSpec((m, dm), lambda i, s: (i, 0)),
                pl.BlockSpec((1, dm, fd), lambda i, s: (s[i], 0, 0)),
                pl.BlockSpec((1, 1, fd), lambda i, s: (s[i], 0, 0)),
                pl.BlockSpec((1, fd, dm), lambda i, s: (s[i], 0, 0)),
                pl.BlockSpec((1, 1, dm), lambda i, s: (s[i], 0, 0)),
            ],
            out_specs=pl.BlockSpec((m, dm), lambda i, s: (i, 0)),
        ),
        out_shape=jax.ShapeDtypeStruct((p_pad, dm), jnp.float32),
    )(meta.reshape(64), xs,
      W1, b1.reshape(ne, 1, fd),
      W2, b2.reshape(ne, 1, dm))

    yg = _gather_sc(ys, pos_b, p)
    y2 = yg.reshape(2, n, dm)

    res = pl.pallas_call(
        _combine_body,
        grid=(nt,),
        in_specs=[
            pl.BlockSpec((mr, dm), lambda t: (t, 0)),
            pl.BlockSpec((mr, dm), lambda t: (t, 0)),
            pl.BlockSpec((mr, 2), lambda t: (t, 0)),
        ],
        out_specs=pl.BlockSpec((mr, dm), lambda t: (t, 0)),
        out_shape=jax.ShapeDtypeStruct((n, dm), jnp.float32),
    )(y2[0], y2[1], w_nk)

    return res.reshape(bsz, seq, dm), laux[0, 0]


# speedup vs baseline: 3.6137x; 1.0019x over previous
import functools

import jax
import jax.numpy as jnp
from jax import lax
from jax.experimental import pallas as pl
from jax.experimental.pallas import tpu as pltpu
from jax.experimental.pallas import tpu_sc as plsc

NW = 32


def _routing_body(nt, ntok, x_ref, tp_ref, wgi_ref, wgt_ref, bg_ref,
                  laux_ref, w_ref, e_ref, r_ref, cnt16_ref, acc_ref, cnt_ref):
    t = pl.program_id(0)
    logits = (jnp.dot(x_ref[...], wgi_ref[...], preferred_element_type=jnp.float32)
              + jnp.dot(tp_ref[...], wgt_ref[...], preferred_element_type=jnp.float32)
              + bg_ref[...])
    mr, ne = logits.shape
    lane = lax.broadcasted_iota(jnp.int32, logits.shape, 1)
    m1 = jnp.max(logits, axis=1, keepdims=True)
    i1 = jnp.min(jnp.where(logits == m1, lane, ne), axis=1, keepdims=True)
    masked = jnp.where(lane == i1, -jnp.inf, logits)
    m2 = jnp.max(masked, axis=1, keepdims=True)
    i2 = jnp.min(jnp.where(masked == m2, lane, ne), axis=1, keepdims=True)
    s = jnp.exp(m2 - m1)
    w1 = 1.0 / (1.0 + s)
    w2 = s / (1.0 + s)
    sel1 = lane == i1
    sel2 = lane == i2

    @pl.when(t == 0)
    def _():
        acc_ref[...] = jnp.zeros_like(acc_ref)
        cnt_ref[...] = jnp.zeros_like(cnt_ref)

    o0 = sel1.astype(jnp.float32)
    o1 = sel2.astype(jnp.float32)
    row = lax.broadcasted_iota(jnp.int32, (mr, mr), 0)
    col = lax.broadcasted_iota(jnp.int32, (mr, mr), 1)
    tril = (col < row).astype(jnp.float32)
    rk0 = jnp.dot(tril, o0, preferred_element_type=jnp.float32)
    rk1 = jnp.dot(tril, o1, preferred_element_type=jnp.float32)
    base0 = cnt_ref[0:1, :].astype(jnp.float32)
    base1 = cnt_ref[1:2, :].astype(jnp.float32)
    r0 = jnp.sum(jnp.where(sel1, rk0 + base0, 0.0), axis=1, keepdims=True)
    r1 = jnp.sum(jnp.where(sel2, rk1 + base1, 0.0), axis=1, keepdims=True)
    r_ref[...] = jnp.concatenate([r0, r1], axis=1).astype(jnp.int32)
    e_ref[...] = jnp.concatenate([i1, i2], axis=1)
    w_ref[...] = jnp.concatenate([w1, w2], axis=1)
    cnt_ref[0:1, :] += jnp.sum(o0, axis=0, keepdims=True).astype(jnp.int32)
    cnt_ref[1:2, :] += jnp.sum(o1, axis=0, keepdims=True).astype(jnp.int32)

    p = jnp.exp(logits - m1)
    p = p / jnp.sum(p, axis=1, keepdims=True)
    acc_ref[0:1, :] += jnp.sum(p, axis=0, keepdims=True)
    acc_ref[1:2, :] += jnp.sum(o0 + o1, axis=0, keepdims=True)

    @pl.when(t == nt - 1)
    def _():
        laux_ref[...] = (jnp.sum(acc_ref[0:1, :] * acc_ref[1:2, :])
                         / (ntok * ntok)).reshape(1, 1)
        cnt16_ref[...] = jnp.concatenate([cnt_ref[0:1, :], cnt_ref[1:2, :]], axis=1)


def _finalize_body(m, ne, cnt_ref, e_ref, r_ref, pos_ref, meta_ref):
    t = pl.program_id(0)
    e_nk = e_ref[...]
    kk = lax.broadcasted_iota(jnp.int32, e_nk.shape, 1)
    lane64 = lax.broadcasted_iota(jnp.int32, (1, 64), 1)
    add = jnp.zeros(e_nk.shape, jnp.int32)
    te = jnp.zeros((1, 64), jnp.int32)
    po = jnp.int32(0)
    btl = jnp.int32(0)
    for e in range(ne):
        c0 = cnt_ref[e]
        c1 = cnt_ref[ne + e]
        pc = ((c0 + c1 + m - 1) // m) * m
        add = add + jnp.where(e_nk == e, po + jnp.where(kk == 1, c0, 0), 0)
        po = po + pc
        btl = btl + pc // m
        te = te + (lane64 >= btl).astype(jnp.int32)
    pos_ref[...] = r_ref[...] + add
    na = btl

    @pl.when(t == 0)
    def _():
        meta_ref[...] = jnp.where(lane64 == 63, na, jnp.minimum(te, ne - 1))


def _dispatch_sc(x, pos_a, p_pad):
    n, dm = x.shape
    nk, nl, sub = pos_a.shape[1:]
    mesh = plsc.VectorSubcoreMesh(core_axis_name="c", subcore_axis_name="s")

    @functools.partial(
        pl.kernel, mesh=mesh,
        out_type=jax.ShapeDtypeStruct((p_pad, dm), jnp.float32),
        scratch_types=[
            pltpu.VMEM(pos_a.shape[1:], jnp.int32),
            pltpu.VMEM((sub, dm), jnp.float32),
            pltpu.VMEM((sub, dm), jnp.float32),
            pltpu.SemaphoreType.DMA,
            pltpu.SemaphoreType.DMA,
            pltpu.SemaphoreType.DMA,
            pltpu.SemaphoreType.DMA,
        ],
    )
    def k(x_hbm, pos_hbm, out_hbm, idx_v, r0, r1, sa, sb, sc_, sd):
        wid = lax.axis_index("s") * 2 + lax.axis_index("c")
        pltpu.sync_copy(pos_hbm.at[wid], idx_v)
        base = wid * nl * sub
        bufs = (r0, r1)
        ssem = (sa, sb)
        wsem = (sc_, sd)
        st = {}
        wts = {}
        st[0] = pltpu.async_copy(x_hbm.at[pl.ds(base, sub)], r0, sa)
        st[1] = pltpu.async_copy(x_hbm.at[pl.ds(base + sub, sub)], r1, sb)
        for l in range(nl):
            buf = bufs[l % 2]
            st[l].wait()
            wts[l] = [pltpu.async_copy(buf, out_hbm.at[idx_v.at[kk, l]],
                                       wsem[l % 2]) for kk in range(nk)]
            if l + 2 < nl:
                for c in wts[l]:
                    c.wait()
                st[l + 2] = pltpu.async_copy(
                    x_hbm.at[pl.ds(base + (l + 2) * sub, sub)], buf, ssem[l % 2])
        for l in range(max(0, nl - 2), nl):
            for c in wts[l]:
                c.wait()

    return k(x, pos_a)


def _gather_sc(ys, pos_b, p):
    dm = ys.shape[1]
    chunk = pos_b.shape[1] * pos_b.shape[2]
    sub = pos_b.shape[2]
    mesh = plsc.VectorSubcoreMesh(core_axis_name="c", subcore_axis_name="s")

    nl = pos_b.shape[1]

    @functools.partial(
        pl.kernel, mesh=mesh,
        out_type=jax.ShapeDtypeStruct((p, dm), jnp.float32),
        scratch_types=[
            pltpu.VMEM(pos_b.shape[1:], jnp.int32),
            pltpu.VMEM((sub, dm), jnp.float32),
            pltpu.VMEM((sub, dm), jnp.float32),
            pltpu.SemaphoreType.DMA,
            pltpu.SemaphoreType.DMA,
            pltpu.SemaphoreType.DMA,
            pltpu.SemaphoreType.DMA,
        ],
    )
    def k(ys_hbm, pos_hbm, out_hbm, idx_v, r0, r1, sa, sb, sc_, sd):
        wid = lax.axis_index("s") * 2 + lax.axis_index("c")
        pltpu.sync_copy(pos_hbm.at[wid], idx_v)
        base = wid * chunk
        bufs = (r0, r1)
        gsem = (sa, sb)
        wsem = (sc_, sd)
        st = {}
        wts = {}
        st[0] = pltpu.async_copy(ys_hbm.at[idx_v.at[0]], r0, sa)
        st[1] = pltpu.async_copy(ys_hbm.at[idx_v.at[1]], r1, sb)
        for l in range(nl):
            buf = bufs[l % 2]
            st[l].wait()
            wts[l] = pltpu.async_copy(
                buf, out_hbm.at[pl.ds(base + l * sub, sub)], wsem[l % 2])
            if l + 2 < nl:
                wts[l].wait()
                st[l + 2] = pltpu.async_copy(
                    ys_hbm.at[idx_v.at[l + 2]], buf, gsem[l % 2])
        for l in range(max(0, nl - 2), nl):
            wts[l].wait()

    return k(ys, pos_b)


def _gffn_body(meta_ref, x_ref, w1_ref, b1_ref, w2_ref, b2_ref, out_ref):
    i = pl.program_id(0)
    na = meta_ref[63]

    @pl.when(i < na)
    def _():
        xb = x_ref[...].astype(jnp.bfloat16)
        w1 = w1_ref[0].astype(jnp.bfloat16)
        h = jnp.dot(xb, w1, preferred_element_type=jnp.float32) + b1_ref[0]
        h = jnp.maximum(h, 0.0).astype(jnp.bfloat16)
        w2 = w2_ref[0].astype(jnp.bfloat16)
        y = jnp.dot(h, w2, preferred_element_type=jnp.float32) + b2_ref[0]
        out_ref[...] = y


def _combine_body(y0_ref, y1_ref, w_ref, out_ref):
    out_ref[...] = (w_ref[:, 0:1] * y0_ref[...]
                    + w_ref[:, 1:2] * y1_ref[...])


def kernel(inputs, task_param, alpha, Wg_in, bg_in, Wg_task, bg_task, W1, b1, W2, b2):
    bsz, seq, dm = inputs.shape
    ne = Wg_in.shape[1]
    fd = W1.shape[2]
    n = bsz * seq
    m = 256
    p = n * 2
    p_pad = p + ne * m
    tf = p_pad // m

    xf = inputs.reshape(n, dm)
    tpf = task_param.reshape(n, dm)
    a = alpha.astype(jnp.float32)
    wgi = (1.0 - a) * Wg_in
    wgt = a * Wg_task
    bg = ((1.0 - a) * bg_in + a * bg_task).reshape(1, ne)

    mr = min(1024, n)
    nt = n // mr
    laux, w_nk, e_nk, r_nk, cnt16 = pl.pallas_call(
        functools.partial(_routing_body, nt, n),
        grid=(nt,),
        in_specs=[
            pl.BlockSpec((mr, dm), lambda t: (t, 0)),
            pl.BlockSpec((mr, dm), lambda t: (t, 0)),
            pl.BlockSpec((dm, ne), lambda t: (0, 0)),
            pl.BlockSpec((dm, ne), lambda t: (0, 0)),
            pl.BlockSpec((1, ne), lambda t: (0, 0)),
        ],
        out_specs=[
            pl.BlockSpec((1, 1), lambda t: (0, 0)),
            pl.BlockSpec((mr, 2), lambda t: (t, 0)),
            pl.BlockSpec((mr, 2), lambda t: (t, 0)),
            pl.BlockSpec((mr, 2), lambda t: (t, 0)),
            pl.BlockSpec((1, 2 * ne), lambda t: (0, 0)),
        ],
        out_shape=[
            jax.ShapeDtypeStruct((1, 1), jnp.float32),
            jax.ShapeDtypeStruct((n, 2), jnp.float32),
            jax.ShapeDtypeStruct((n, 2), jnp.int32),
            jax.ShapeDtypeStruct((n, 2), jnp.int32),
            jax.ShapeDtypeStruct((1, 2 * ne), jnp.int32),
        ],
        scratch_shapes=[pltpu.VMEM((2, ne), jnp.float32),
                        pltpu.VMEM((2, ne), jnp.int32)],
    )(xf, tpf, wgi, wgt, bg)

    pos_nk, meta = pl.pallas_call(
        functools.partial(_finalize_body, m, ne),
        grid_spec=pltpu.PrefetchScalarGridSpec(
            num_scalar_prefetch=1,
            grid=(nt,),
            in_specs=[
                pl.BlockSpec((mr, 2), lambda t, c: (t, 0)),
                pl.BlockSpec((mr, 2), lambda t, c: (t, 0)),
            ],
            out_specs=[
                pl.BlockSpec((mr, 2), lambda t, c: (t, 0)),
                pl.BlockSpec((1, 64), lambda t, c: (0, 0)),
            ],
        ),
        out_shape=[
            jax.ShapeDtypeStruct((n, 2), jnp.int32),
            jax.ShapeDtypeStruct((1, 64), jnp.int32),
        ],
    )(cnt16.reshape(2 * ne), e_nk, r_nk)

    posk = jnp.transpose(pos_nk, (1, 0)).reshape(p)
    pos_b = posk.reshape(NW, -1, 32)
    pos_a = jnp.transpose(posk.reshape(2, NW, -1, 32), (1, 0, 2, 3))

    xs = _dispatch_sc(xf, pos_a, p_pad)

    ys = pl.pallas_call(
        _gffn_body,
        grid_spec=pltpu.PrefetchScalarGridSpec(
            num_scalar_prefetch=1,
            grid=(tf,),
            in_specs=[
                pl.BlockSpec((m, dm), lambda i, s: (i, 0)),
                pl.BlockSpec((1, dm, fd), lambda i, s: (s[i], 0, 0)),
                pl.BlockSpec((1, 1, fd), lambda i, s: (s[i], 0, 0)),
                pl.BlockSpec((1, fd, dm), lambda i, s: (s[i], 0, 0)),
                pl.BlockSpec((1, 1, dm), lambda i, s: (s[i], 0, 0)),
            ],
            out_specs=pl.BlockSpec((m, dm), lambda i, s: (i, 0)),
        ),
        out_shape=jax.ShapeDtypeStruct((p_pad, dm), jnp.float32),
    )(meta.reshape(64), xs,
      W1, b1.reshape(ne, 1, fd),
      W2, b2.reshape(ne, 1, dm))

    yg = _gather_sc(ys, pos_b, p)
    y2 = yg.reshape(2, n, dm)

    res = pl.pallas_call(
        _combine_body,
        grid=(nt,),
        in_specs=[
            pl.BlockSpec((mr, dm), lambda t: (t, 0)),
            pl.BlockSpec((mr, dm), lambda t: (t, 0)),
            pl.BlockSpec((mr, 2), lambda t: (t, 0)),
        ],
        out_specs=pl.BlockSpec((mr, dm), lambda t: (t, 0)),
        out_shape=jax.ShapeDtypeStruct((n, dm), jnp.float32),
    )(y2[0], y2[1], w_nk)

    return res.reshape(bsz, seq, dm), laux[0, 0]


# speedup vs baseline: 3.6639x; 1.0139x over previous
import functools

import jax
import jax.numpy as jnp
from jax import lax
from jax.experimental import pallas as pl
from jax.experimental.pallas import tpu as pltpu
from jax.experimental.pallas import tpu_sc as plsc

NW = 32


def _routing_body(nt, ntok, m, x_ref, tp_ref, wgi_ref, wgt_ref, bg_ref,
                  laux_ref, w_ref, pos_ref, meta_ref,
                  acc_ref, cnt_ref, e_scr, r_scr):
    t = pl.program_id(0)
    mr = x_ref.shape[0]
    logits = (jnp.dot(x_ref[...], wgi_ref[...], preferred_element_type=jnp.float32)
              + jnp.dot(tp_ref[...], wgt_ref[...], preferred_element_type=jnp.float32)
              + bg_ref[...])
    mr, ne = logits.shape
    lane = lax.broadcasted_iota(jnp.int32, logits.shape, 1)
    m1 = jnp.max(logits, axis=1, keepdims=True)
    i1 = jnp.min(jnp.where(logits == m1, lane, ne), axis=1, keepdims=True)
    masked = jnp.where(lane == i1, -jnp.inf, logits)
    m2 = jnp.max(masked, axis=1, keepdims=True)
    i2 = jnp.min(jnp.where(masked == m2, lane, ne), axis=1, keepdims=True)
    s = jnp.exp(m2 - m1)
    w1 = 1.0 / (1.0 + s)
    w2 = s / (1.0 + s)
    sel1 = lane == i1
    sel2 = lane == i2

    @pl.when(t == 0)
    def _():
        acc_ref[...] = jnp.zeros_like(acc_ref)
        cnt_ref[...] = jnp.zeros_like(cnt_ref)

    o0 = sel1.astype(jnp.float32)
    o1 = sel2.astype(jnp.float32)
    row = lax.broadcasted_iota(jnp.int32, (mr, mr), 0)
    col = lax.broadcasted_iota(jnp.int32, (mr, mr), 1)
    tril = (col < row).astype(jnp.float32)
    rk0 = jnp.dot(tril, o0, preferred_element_type=jnp.float32)
    rk1 = jnp.dot(tril, o1, preferred_element_type=jnp.float32)
    base0 = cnt_ref[0:1, :].astype(jnp.float32)
    base1 = cnt_ref[1:2, :].astype(jnp.float32)
    r0 = jnp.sum(jnp.where(sel1, rk0 + base0, 0.0), axis=1, keepdims=True)
    r1 = jnp.sum(jnp.where(sel2, rk1 + base1, 0.0), axis=1, keepdims=True)
    r_scr[pl.ds(t * mr, mr), :] = jnp.concatenate([r0, r1], axis=1).astype(jnp.int32)
    e_scr[pl.ds(t * mr, mr), :] = jnp.concatenate([i1, i2], axis=1)
    w_ref[...] = jnp.concatenate([w1, w2], axis=1)
    cnt_ref[0:1, :] += jnp.sum(o0, axis=0, keepdims=True).astype(jnp.int32)
    cnt_ref[1:2, :] += jnp.sum(o1, axis=0, keepdims=True).astype(jnp.int32)

    p = jnp.exp(logits - m1)
    p = p / jnp.sum(p, axis=1, keepdims=True)
    acc_ref[0:1, :] += jnp.sum(p, axis=0, keepdims=True)
    acc_ref[1:2, :] += jnp.sum(o0 + o1, axis=0, keepdims=True)

    @pl.when(t == nt - 1)
    def _():
        laux_ref[...] = (jnp.sum(acc_ref[0:1, :] * acc_ref[1:2, :])
                         / (ntok * ntok)).reshape(1, 1)
        ne2 = bg_ref.shape[1]
        mlog2 = m.bit_length() - 1
        e_all = e_scr[...]
        kk2 = lax.broadcasted_iota(jnp.int32, e_all.shape, 1)
        lane64 = lax.broadcasted_iota(jnp.int32, (1, 64), 1)
        add = jnp.zeros(e_all.shape, jnp.int32)
        te = jnp.zeros((1, 64), jnp.int32)
        po = jnp.zeros((1, 1), jnp.int32)
        btl = jnp.zeros((1, 1), jnp.int32)
        for e in range(ne2):
            c0 = cnt_ref[0:1, e:e + 1]
            c1 = cnt_ref[1:2, e:e + 1]
            nt_e = lax.shift_right_logical(c0 + c1 + m - 1, mlog2)
            add = add + jnp.where(e_all == e,
                                  po + jnp.where(kk2 == 1, c0, 0), 0)
            po = po + lax.shift_left(nt_e, mlog2)
            btl = btl + nt_e
            te = te + (lane64 >= btl).astype(jnp.int32)
        pos_ref[...] = r_scr[...] + add
        meta_ref[...] = jnp.where(lane64 == 63, btl, jnp.minimum(te, ne2 - 1))


def _dispatch_sc(x, pos_a, p_pad):
    n, dm = x.shape
    nk, nl, sub = pos_a.shape[1:]
    mesh = plsc.VectorSubcoreMesh(core_axis_name="c", subcore_axis_name="s")

    @functools.partial(
        pl.kernel, mesh=mesh,
        out_type=jax.ShapeDtypeStruct((p_pad, dm), jnp.float32),
        scratch_types=[
            pltpu.VMEM(pos_a.shape[1:], jnp.int32),
            pltpu.VMEM((sub, dm), jnp.float32),
            pltpu.VMEM((sub, dm), jnp.float32),
            pltpu.SemaphoreType.DMA,
            pltpu.SemaphoreType.DMA,
            pltpu.SemaphoreType.DMA,
            pltpu.SemaphoreType.DMA,
        ],
    )
    def k(x_hbm, pos_hbm, out_hbm, idx_v, r0, r1, sa, sb, sc_, sd):
        wid = lax.axis_index("s") * 2 + lax.axis_index("c")
        pltpu.sync_copy(pos_hbm.at[wid], idx_v)
        base = wid * nl * sub
        bufs = (r0, r1)
        ssem = (sa, sb)
        wsem = (sc_, sd)
        st = {}
        wts = {}
        st[0] = pltpu.async_copy(x_hbm.at[pl.ds(base, sub)], r0, sa)
        st[1] = pltpu.async_copy(x_hbm.at[pl.ds(base + sub, sub)], r1, sb)
        for l in range(nl):
            buf = bufs[l % 2]
            st[l].wait()
            wts[l] = [pltpu.async_copy(buf, out_hbm.at[idx_v.at[kk, l]],
                                       wsem[l % 2]) for kk in range(nk)]
            if l + 2 < nl:
                for c in wts[l]:
                    c.wait()
                st[l + 2] = pltpu.async_copy(
                    x_hbm.at[pl.ds(base + (l + 2) * sub, sub)], buf, ssem[l % 2])
        for l in range(max(0, nl - 2), nl):
            for c in wts[l]:
                c.wait()

    return k(x, pos_a)


def _gather_sc(ys, pos_b, p):
    dm = ys.shape[1]
    chunk = pos_b.shape[1] * pos_b.shape[2]
    sub = pos_b.shape[2]
    mesh = plsc.VectorSubcoreMesh(core_axis_name="c", subcore_axis_name="s")

    nl = pos_b.shape[1]

    @functools.partial(
        pl.kernel, mesh=mesh,
        out_type=jax.ShapeDtypeStruct((p, dm), jnp.float32),
        scratch_types=[
            pltpu.VMEM(pos_b.shape[1:], jnp.int32),
            pltpu.VMEM((sub, dm), jnp.float32),
            pltpu.VMEM((sub, dm), jnp.float32),
            pltpu.SemaphoreType.DMA,
            pltpu.SemaphoreType.DMA,
            pltpu.SemaphoreType.DMA,
            pltpu.SemaphoreType.DMA,
        ],
    )
    def k(ys_hbm, pos_hbm, out_hbm, idx_v, r0, r1, sa, sb, sc_, sd):
        wid = lax.axis_index("s") * 2 + lax.axis_index("c")
        pltpu.sync_copy(pos_hbm.at[wid], idx_v)
        base = wid * chunk
        bufs = (r0, r1)
        gsem = (sa, sb)
        wsem = (sc_, sd)
        st = {}
        wts = {}
        st[0] = pltpu.async_copy(ys_hbm.at[idx_v.at[0]], r0, sa)
        st[1] = pltpu.async_copy(ys_hbm.at[idx_v.at[1]], r1, sb)
        for l in range(nl):
            buf = bufs[l % 2]
            st[l].wait()
            wts[l] = pltpu.async_copy(
                buf, out_hbm.at[pl.ds(base + l * sub, sub)], wsem[l % 2])
            if l + 2 < nl:
                wts[l].wait()
                st[l + 2] = pltpu.async_copy(
                    ys_hbm.at[idx_v.at[l + 2]], buf, gsem[l % 2])
        for l in range(max(0, nl - 2), nl):
            wts[l].wait()

    return k(ys, pos_b)


def _gffn_body(meta_ref, x_ref, w1_ref, b1_ref, w2_ref, b2_ref, out_ref):
    i = pl.program_id(0)
    na = meta_ref[63]

    @pl.when(i < na)
    def _():
        xb = x_ref[...].astype(jnp.bfloat16)
        w1 = w1_ref[0].astype(jnp.bfloat16)
        h = jnp.dot(xb, w1, preferred_element_type=jnp.float32) + b1_ref[0]
        h = jnp.maximum(h, 0.0).astype(jnp.bfloat16)
        w2 = w2_ref[0].astype(jnp.bfloat16)
        y = jnp.dot(h, w2, preferred_element_type=jnp.float32) + b2_ref[0]
        out_ref[...] = y


def _combine_body(y0_ref, y1_ref, w_ref, out_ref):
    out_ref[...] = (w_ref[:, 0:1] * y0_ref[...]
                    + w_ref[:, 1:2] * y1_ref[...])


def kernel(inputs, task_param, alpha, Wg_in, bg_in, Wg_task, bg_task, W1, b1, W2, b2):
    bsz, seq, dm = inputs.shape
    ne = Wg_in.shape[1]
    fd = W1.shape[2]
    n = bsz * seq
    m = 256
    p = n * 2
    p_pad = p + ne * m
    tf = p_pad // m

    xf = inputs.reshape(n, dm)
    tpf = task_param.reshape(n, dm)
    a = alpha.astype(jnp.float32)
    wgi = (1.0 - a) * Wg_in
    wgt = a * Wg_task
    bg = ((1.0 - a) * bg_in + a * bg_task).reshape(1, ne)

    mr = min(1024, n)
    nt = n // mr
    laux, w_nk, pos_nk, meta = pl.pallas_call(
        functools.partial(_routing_body, nt, n, m),
        grid=(nt,),
        in_specs=[
            pl.BlockSpec((mr, dm), lambda t: (t, 0)),
            pl.BlockSpec((mr, dm), lambda t: (t, 0)),
            pl.BlockSpec((dm, ne), lambda t: (0, 0)),
            pl.BlockSpec((dm, ne), lambda t: (0, 0)),
            pl.BlockSpec((1, ne), lambda t: (0, 0)),
        ],
        out_specs=[
            pl.BlockSpec((1, 1), lambda t: (0, 0)),
            pl.BlockSpec((mr, 2), lambda t: (t, 0)),
            pl.BlockSpec((n, 2), lambda t: (0, 0)),
            pl.BlockSpec((1, 64), lambda t: (0, 0)),
        ],
        out_shape=[
            jax.ShapeDtypeStruct((1, 1), jnp.float32),
            jax.ShapeDtypeStruct((n, 2), jnp.float32),
            jax.ShapeDtypeStruct((n, 2), jnp.int32),
            jax.ShapeDtypeStruct((1, 64), jnp.int32),
        ],
        scratch_shapes=[pltpu.VMEM((2, ne), jnp.float32),
                        pltpu.VMEM((2, ne), jnp.int32),
                        pltpu.VMEM((n, 2), jnp.int32),
                        pltpu.VMEM((n, 2), jnp.int32)],
    )(xf, tpf, wgi, wgt, bg)

    posk = jnp.transpose(pos_nk, (1, 0)).reshape(p)
    pos_b = posk.reshape(NW, -1, 32)
    pos_a = jnp.transpose(posk.reshape(2, NW, -1, 32), (1, 0, 2, 3))

    xs = _dispatch_sc(xf, pos_a, p_pad)

    ys = pl.pallas_call(
        _gffn_body,
        grid_spec=pltpu.PrefetchScalarGridSpec(
            num_scalar_prefetch=1,
            grid=(tf,),
            in_specs=[
                pl.BlockSpec((m, dm), lambda i, s: (i, 0)),
                pl.BlockSpec((1, dm, fd), lambda i, s: (s[i], 0, 0)),
                pl.BlockSpec((1, 1, fd), lambda i, s: (s[i], 0, 0)),
                pl.BlockSpec((1, fd, dm), lambda i, s: (s[i], 0, 0)),
                pl.BlockSpec((1, 1, dm), lambda i, s: (s[i], 0, 0)),
            ],
            out_specs=pl.BlockSpec((m, dm), lambda i, s: (i, 0)),
        ),
        out_shape=jax.ShapeDtypeStruct((p_pad, dm), jnp.float32),
    )(meta.reshape(64), xs,
      W1, b1.reshape(ne, 1, fd),
      W2, b2.reshape(ne, 1, dm))

    yg = _gather_sc(ys, pos_b, p)
    y2 = yg.reshape(2, n, dm)

    res = pl.pallas_call(
        _combine_body,
        grid=(nt,),
        in_specs=[
            pl.BlockSpec((mr, dm), lambda t: (t, 0)),
            pl.BlockSpec((mr, dm), lambda t: (t, 0)),
            pl.BlockSpec((mr, 2), lambda t: (t, 0)),
        ],
        out_specs=pl.BlockSpec((mr, dm), lambda t: (t, 0)),
        out_shape=jax.ShapeDtypeStruct((n, dm), jnp.float32),
    )(y2[0], y2[1], w_nk)

    return res.reshape(bsz, seq, dm), laux[0, 0]
